# baseline (speedup 1.0000x reference)
import dataclasses
import functools

import jax
import jax.numpy as jnp
from jax import lax
from jax.experimental import pallas as pl
from jax.experimental.pallas import tpu as pltpu
from jax.experimental.pallas import tpu_sc as plsc

N = 10000
NPAD = 10240
E = 160000
EPAD = 163840
DIN = 256
DH = 32
P = 12
K2 = 2 * DH
NSLICE = 6
BN = 512
NBLK = NPAD // BN


def _mesh_():
    return plsc.VectorSubcoreMesh(core_axis_name="core", subcore_axis_name="subcore",
                                  num_cores=2, num_subcores=16)


def _sc_params():
    cp = pltpu.CompilerParams(use_tc_tiling_on_sc=True)
    if "needs_layout_passes" in pltpu.CompilerParams.__dataclass_fields__:
        cp = dataclasses.replace(cp, needs_layout_passes=False)
    return cp


def _sc_deg(dst3, w3):
    @functools.partial(
        pl.kernel,
        out_type=jax.ShapeDtypeStruct((2, NPAD), jnp.float32),
        mesh=_mesh_(),
        scratch_types=[
            pltpu.VMEM((40, 128), jnp.int32),
            pltpu.VMEM((40, 128), jnp.float32),
            pltpu.VMEM((640,), jnp.float32),
            pltpu.VMEM_SHARED((NPAD,), jnp.float32),
        ],
        compiler_params=_sc_params(),
    )
    def k(dst_hbm, w_hbm, out_hbm, idx_v, w_v, zbuf, deg_sh):
        c = lax.axis_index("core")
        t = lax.axis_index("subcore")
        wid = c * 16 + t

        @pl.loop(0, 40)
        def _(i):
            zbuf[pl.ds(i * 16, 16)] = jnp.zeros((16,), jnp.float32)

        pltpu.sync_copy(zbuf, deg_sh.at[pl.ds(t * 640, 640)])
        plsc.subcore_barrier()

        pltpu.sync_copy(dst_hbm.at[wid], idx_v)
        pltpu.sync_copy(w_hbm.at[wid], w_v)

        @pl.loop(0, 40)
        def _(j):
            pltpu.sync_copy(w_v.at[j], deg_sh.at[idx_v.at[j]], add=True)

        plsc.subcore_barrier()
        pltpu.sync_copy(deg_sh.at[pl.ds(t * 640, 640)],
                        out_hbm.at[c, pl.ds(t * 640, 640)])

    return k(dst3, w3)


def _sc_norm(src3, dst3, w3, dis):
    @functools.partial(
        pl.kernel,
        out_type=jax.ShapeDtypeStruct((32, 40, 128), jnp.float32),
        mesh=_mesh_(),
        scratch_types=[
            pltpu.VMEM((40, 128), jnp.int32),
            pltpu.VMEM((40, 128), jnp.int32),
            pltpu.VMEM((40, 128), jnp.float32),
            pltpu.VMEM((NPAD,), jnp.float32),
        ],
        compiler_params=_sc_params(),
    )
    def k(src_hbm, dst_hbm, w_hbm, dis_hbm, out_hbm, sv, dv, wv, disv):
        c = lax.axis_index("core")
        t = lax.axis_index("subcore")
        wid = c * 16 + t

        pltpu.sync_copy(dis_hbm, disv)
        pltpu.sync_copy(src_hbm.at[wid], sv)
        pltpu.sync_copy(dst_hbm.at[wid], dv)
        pltpu.sync_copy(w_hbm.at[wid], wv)

        @pl.loop(0, 40)
        def _(j):
            for v in range(8):
                sl = pl.ds(v * 16, 16)
                s16 = sv[j, sl]
                d16 = dv[j, sl]
                wv[j, sl] = (plsc.load_gather(disv, [s16]) * wv[j, sl]
                             * plsc.load_gather(disv, [d16]))

        pltpu.sync_copy(wv, out_hbm.at[wid])

    return k(src3, dst3, w3, dis)


def _sc_agg(proj6, src4, dst4, normp):
    @functools.partial(
        pl.kernel,
        out_type=jax.ShapeDtypeStruct((NSLICE, NPAD, 128), jnp.float32),
        mesh=_mesh_(),
        scratch_types=[
            pltpu.VMEM((8, 128), jnp.int32),
            pltpu.VMEM((8, 128), jnp.int32),
            pltpu.VMEM((8, 128), jnp.float32),
            pltpu.VMEM((128, 128), jnp.float32),
            pltpu.VMEM((128, 128), jnp.float32),
            pltpu.SemaphoreType.DMA,
            pltpu.SemaphoreType.DMA,
            pltpu.SemaphoreType.DMA,
            pltpu.SemaphoreType.DMA,
            pltpu.VMEM_SHARED((NPAD, 128), jnp.float32),
        ],
        compiler_params=_sc_params(),
    )
    def k(proj_hbm, src_hbm, dst_hbm, norm_hbm, out_hbm,
          srcb, dstb, normb, g0, g1,
          sg0, sg1, ss0, ss1, ysh):
        c = lax.axis_index("core")
        t = lax.axis_index("subcore")

        gbufs = (g0, g1)
        gsems = (sg0, sg1)
        ssems = (ss0, ss1)

        @pl.loop(0, 3)
        def _(sp):
            s = c * 3 + sp

            @pl.loop(0, 128)
            def _(i):
                for v in range(8):
                    g0[i, pl.ds(v * 16, 16)] = jnp.zeros((16,), jnp.float32)

            @pl.loop(0, 5)
            def _(i):
                pltpu.sync_copy(g0, ysh.at[pl.ds(t * 640 + i * 128, 128)])

            plsc.subcore_barrier()

            table = proj_hbm.at[s]

            @pl.loop(0, 10)
            def _(b):
                pltpu.sync_copy(src_hbm.at[t, pl.ds(b * 8, 8)], srcb)
                for h in range(2):
                    pltpu.async_copy(table.at[srcb.at[h]], gbufs[h], gsems[h])
                pltpu.sync_copy(dst_hbm.at[t, pl.ds(b * 8, 8)], dstb)
                pltpu.sync_copy(norm_hbm.at[t, pl.ds(b * 8, 8)], normb)

                @pl.loop(0, 4)
                def _(q):
                    for h in range(2):
                        gb, sgh, ssh = gbufs[h], gsems[h], ssems[h]
                        hp = (h + 1) % 2
                        j = q * 2 + h
                        pltpu.make_async_copy(table.at[srcb.at[j]], gb, sgh).wait()

                        @pl.loop(0, 8)
                        def _(i16):
                            nv16 = normb[j, pl.ds(i16 * 16, 16)]
                            for l in range(16):
                                i = i16 * 16 + l
                                nv = nv16[l]
                                for v in range(8):
                                    gb[i, pl.ds(v * 16, 16)] = (
                                        gb[i, pl.ds(v * 16, 16)] * nv)

                        pltpu.async_copy(gb, ysh.at[dstb.at[j]], ssh, add=True)

                        jp = j - 1

                        @pl.when(jnp.logical_and(jp >= 0, jp <= 5))
                        def _():
                            pltpu.make_async_copy(
                                gbufs[hp], ysh.at[dstb.at[jp]], ssems[hp]).wait()
                            pltpu.async_copy(table.at[srcb.at[jp + 2]],
                                             gbufs[hp], gsems[hp])

                for h in range(2):
                    pltpu.make_async_copy(gbufs[h], ysh.at[dstb.at[6 + h]],
                                          ssems[h]).wait()

            plsc.subcore_barrier()

            @pl.loop(0, 5)
            def _(i):
                pltpu.sync_copy(ysh.at[pl.ds(t * 640 + i * 128, 128)],
                                out_hbm.at[s, pl.ds(t * 640 + i * 128, 128)])

            plsc.subcore_barrier()

    return k(proj6, src4, dst4, normp)


def _tc_proj(xt, m):
    def body(xa_ref, xb_ref, m_ref, o_ref):
        a = jnp.dot(xa_ref[0], m_ref[...], preferred_element_type=jnp.float32)
        b = jnp.dot(xb_ref[0], m_ref[...], preferred_element_type=jnp.float32)
        o_ref[0] = jnp.concatenate([a, b], axis=1)

    return pl.pallas_call(
        body,
        grid=(NSLICE, NBLK),
        in_specs=[
            pl.BlockSpec((1, BN, DIN), lambda s, i: (2 * s, i, 0)),
            pl.BlockSpec((1, BN, DIN), lambda s, i: (2 * s + 1, i, 0)),
            pl.BlockSpec((DIN, K2), lambda s, i: (0, 0)),
        ],
        out_specs=pl.BlockSpec((1, BN, 128), lambda s, i: (s, i, 0)),
        out_shape=jax.ShapeDtypeStruct((NSLICE, NPAD, 128), jnp.float32),
    )(xt, xt, m)


def _tc_final(yagg, proj6, invdeg, bias128, probs2, W_out, b_out):
    def body(y_ref, p_ref, d_ref, b_ref, pr_ref, wo_ref, bo_ref, o_ref):
        dq = d_ref[...]
        acc = jnp.zeros((BN, DH), jnp.float32)
        for s in range(NSLICE):
            tfull = y_ref[s] + dq * p_ref[s] + b_ref[...]
            for half in range(2):
                p = 2 * s + half
                z = jax.nn.sigmoid(tfull[:, half * K2:half * K2 + DH])
                ht = jnp.tanh(tfull[:, half * K2 + DH:half * K2 + 2 * DH])
                acc = acc + pr_ref[0, p] * ((1.0 - z) * ht)
        o_ref[...] = jnp.dot(jnp.maximum(acc, 0.0), wo_ref[...],
                             preferred_element_type=jnp.float32) + bo_ref[...]

    return pl.pallas_call(
        body,
        grid=(NBLK,),
        in_specs=[
            pl.BlockSpec((NSLICE, BN, 128), lambda i: (0, i, 0)),
            pl.BlockSpec((NSLICE, BN, 128), lambda i: (0, i, 0)),
            pl.BlockSpec((BN, 1), lambda i: (i, 0)),
            pl.BlockSpec((1, 128), lambda i: (0, 0)),
            pl.BlockSpec((1, P), lambda i: (0, 0), memory_space=pltpu.SMEM),
            pl.BlockSpec((DH, P), lambda i: (0, 0)),
            pl.BlockSpec((1, P), lambda i: (0, 0)),
        ],
        out_specs=pl.BlockSpec((BN, P), lambda i: (i, 0)),
        out_shape=jax.ShapeDtypeStruct((NPAD, P), jnp.float32),
    )(yagg, proj6, invdeg, bias128, probs2, W_out, b_out)


def kernel(x, edge_index, edge_attr, Wz, bz, lz_W, lz_b, Wr, br, lr_W, lr_b,
           Wh, bh, lh_W, lh_b, att, W_out, b_out):
    Lz = lz_W[:DH]
    Lh = lh_W[:DH]
    M = jnp.concatenate([Wz @ Lz, Wh @ Lh], axis=1)
    cz = bz @ Lz + lz_b
    ch = bh @ Lh + lh_b
    bias128 = jnp.concatenate([cz, ch, cz, ch]).reshape(1, 128)
    probs2 = jax.nn.softmax(att).reshape(1, P)

    src = edge_index[0]
    dst = edge_index[1]
    pad = EPAD - E
    srcp = jnp.concatenate([src, jnp.zeros((pad,), jnp.int32)])
    dstp = jnp.concatenate([dst, jnp.zeros((pad,), jnp.int32)])
    wp = jnp.concatenate([edge_attr, jnp.zeros((pad,), jnp.float32)])
    src3 = srcp.reshape(32, 40, 128)
    dst3 = dstp.reshape(32, 40, 128)
    w3 = wp.reshape(32, 40, 128)

    degp = _sc_deg(dst3, w3)
    deg = degp[0] + degp[1] + 1.0
    dis = lax.rsqrt(deg)
    invdeg = (1.0 / deg).reshape(NPAD, 1)

    normp = _sc_norm(src3, dst3, w3, dis).reshape(16, 80, 128)

    xt = jnp.transpose(x, (2, 0, 1))
    proj6 = _tc_proj(xt, M)

    yagg = _sc_agg(proj6, srcp.reshape(16, 80, 128),
                   dstp.reshape(16, 80, 128), normp)

    out = _tc_final(yagg, proj6, invdeg, bias128, probs2, W_out,
                    b_out.reshape(1, P))
    return out[:N]


# speedup vs baseline: 1.0904x; 1.0904x over previous
import dataclasses
import functools

import jax
import jax.numpy as jnp
from jax import lax
from jax.experimental import pallas as pl
from jax.experimental.pallas import tpu as pltpu
from jax.experimental.pallas import tpu_sc as plsc

N = 10000
NPAD = 10240
E = 160000
EPAD = 163840
DIN = 256
DH = 32
P = 12
K2 = 2 * DH
NSLICE = 6
BN = 512
NBLK = NPAD // BN


def _mesh_():
    return plsc.VectorSubcoreMesh(core_axis_name="core", subcore_axis_name="subcore",
                                  num_cores=2, num_subcores=16)


def _sc_params():
    cp = pltpu.CompilerParams(use_tc_tiling_on_sc=True)
    if "needs_layout_passes" in pltpu.CompilerParams.__dataclass_fields__:
        cp = dataclasses.replace(cp, needs_layout_passes=False)
    return cp


def _sc_deg(dst3, w3):
    @functools.partial(
        pl.kernel,
        out_type=jax.ShapeDtypeStruct((2, NPAD), jnp.float32),
        mesh=_mesh_(),
        scratch_types=[
            pltpu.VMEM((40, 128), jnp.int32),
            pltpu.VMEM((40, 128), jnp.float32),
            pltpu.VMEM((640,), jnp.float32),
            pltpu.VMEM_SHARED((NPAD,), jnp.float32),
        ],
        compiler_params=_sc_params(),
    )
    def k(dst_hbm, w_hbm, out_hbm, idx_v, w_v, zbuf, deg_sh):
        c = lax.axis_index("core")
        t = lax.axis_index("subcore")
        wid = c * 16 + t

        @pl.loop(0, 40)
        def _(i):
            zbuf[pl.ds(i * 16, 16)] = jnp.zeros((16,), jnp.float32)

        pltpu.sync_copy(zbuf, deg_sh.at[pl.ds(t * 640, 640)])
        plsc.subcore_barrier()

        pltpu.sync_copy(dst_hbm.at[wid], idx_v)
        pltpu.sync_copy(w_hbm.at[wid], w_v)

        @pl.loop(0, 40)
        def _(j):
            pltpu.sync_copy(w_v.at[j], deg_sh.at[idx_v.at[j]], add=True)

        plsc.subcore_barrier()
        pltpu.sync_copy(deg_sh.at[pl.ds(t * 640, 640)],
                        out_hbm.at[c, pl.ds(t * 640, 640)])

    return k(dst3, w3)


def _sc_norm(src3, dst3, w3, dis):
    @functools.partial(
        pl.kernel,
        out_type=jax.ShapeDtypeStruct((32, 40, 128), jnp.float32),
        mesh=_mesh_(),
        scratch_types=[
            pltpu.VMEM((40, 128), jnp.int32),
            pltpu.VMEM((40, 128), jnp.int32),
            pltpu.VMEM((40, 128), jnp.float32),
            pltpu.VMEM((NPAD,), jnp.float32),
        ],
        compiler_params=_sc_params(),
    )
    def k(src_hbm, dst_hbm, w_hbm, dis_hbm, out_hbm, sv, dv, wv, disv):
        c = lax.axis_index("core")
        t = lax.axis_index("subcore")
        wid = c * 16 + t

        pltpu.sync_copy(dis_hbm, disv)
        pltpu.sync_copy(src_hbm.at[wid], sv)
        pltpu.sync_copy(dst_hbm.at[wid], dv)
        pltpu.sync_copy(w_hbm.at[wid], wv)

        @pl.loop(0, 40)
        def _(j):
            for v in range(8):
                sl = pl.ds(v * 16, 16)
                s16 = sv[j, sl]
                d16 = dv[j, sl]
                wv[j, sl] = (plsc.load_gather(disv, [s16]) * wv[j, sl]
                             * plsc.load_gather(disv, [d16]))

        pltpu.sync_copy(wv, out_hbm.at[wid])

    return k(src3, dst3, w3, dis)


def _sc_agg(proj6, src4, dst4, normp):
    @functools.partial(
        pl.kernel,
        out_type=jax.ShapeDtypeStruct((NSLICE, NPAD, 128), jnp.float32),
        mesh=_mesh_(),
        scratch_types=[
            pltpu.VMEM((40, 32), jnp.int32),
            pltpu.VMEM((40, 32), jnp.int32),
            pltpu.VMEM((40, 32), jnp.float32),
            pltpu.VMEM((32, 128), jnp.float32),
            pltpu.VMEM((32, 128), jnp.float32),
            pltpu.VMEM((32, 128), jnp.float32),
            pltpu.VMEM((32, 128), jnp.float32),
            pltpu.VMEM((32, 128), jnp.float32),
            pltpu.VMEM((32, 128), jnp.float32),
            pltpu.VMEM((32, 128), jnp.float32),
            pltpu.VMEM((32, 128), jnp.float32),
            pltpu.SemaphoreType.DMA,
            pltpu.SemaphoreType.DMA,
            pltpu.SemaphoreType.DMA,
            pltpu.SemaphoreType.DMA,
            pltpu.SemaphoreType.DMA,
            pltpu.SemaphoreType.DMA,
            pltpu.SemaphoreType.DMA,
            pltpu.SemaphoreType.DMA,
            pltpu.SemaphoreType.DMA,
            pltpu.SemaphoreType.DMA,
            pltpu.SemaphoreType.DMA,
            pltpu.SemaphoreType.DMA,
            pltpu.SemaphoreType.DMA,
            pltpu.SemaphoreType.DMA,
            pltpu.SemaphoreType.DMA,
            pltpu.SemaphoreType.DMA,
            pltpu.VMEM_SHARED((NPAD, 128), jnp.float32),
        ],
        compiler_params=_sc_params(),
    )
    def k(proj_hbm, src_hbm, dst_hbm, norm_hbm, out_hbm,
          srcb, dstb, normb, g0, g1, g2, g3, g4, g5, g6, g7,
          sg0, sg1, sg2, sg3, sg4, sg5, sg6, sg7,
          ss0, ss1, ss2, ss3, ss4, ss5, ss6, ss7, ysh):
        c = lax.axis_index("core")
        t = lax.axis_index("subcore")

        gbufs = (g0, g1, g2, g3, g4, g5, g6, g7)
        gsems = (sg0, sg1, sg2, sg3, sg4, sg5, sg6, sg7)
        ssems = (ss0, ss1, ss2, ss3, ss4, ss5, ss6, ss7)

        @pl.loop(0, 3)
        def _(sp):
            s = c * 3 + sp

            @pl.loop(0, 32)
            def _(i):
                for v in range(8):
                    g0[i, pl.ds(v * 16, 16)] = jnp.zeros((16,), jnp.float32)

            @pl.loop(0, 20)
            def _(i):
                pltpu.sync_copy(g0, ysh.at[pl.ds(t * 640 + i * 32, 32)])

            plsc.subcore_barrier()

            table = proj_hbm.at[s]

            @pl.loop(0, 8)
            def _(b):
                pltpu.sync_copy(src_hbm.at[t, pl.ds(b * 40, 40)], srcb)
                for h in range(8):
                    pltpu.async_copy(table.at[srcb.at[h]], gbufs[h], gsems[h])
                pltpu.sync_copy(dst_hbm.at[t, pl.ds(b * 40, 40)], dstb)
                pltpu.sync_copy(norm_hbm.at[t, pl.ds(b * 40, 40)], normb)

                @pl.loop(0, 5)
                def _(q):
                    for h in range(8):
                        gb, sgh, ssh = gbufs[h], gsems[h], ssems[h]
                        hp = (h + 7) % 8
                        j = q * 8 + h
                        pltpu.make_async_copy(table.at[srcb.at[j]], gb, sgh).wait()

                        @pl.loop(0, 2)
                        def _(i16):
                            nv16 = normb[j, pl.ds(i16 * 16, 16)]
                            for l in range(16):
                                i = i16 * 16 + l
                                nv = nv16[l]
                                for v in range(8):
                                    gb[i, pl.ds(v * 16, 16)] = (
                                        gb[i, pl.ds(v * 16, 16)] * nv)

                        pltpu.async_copy(gb, ysh.at[dstb.at[j]], ssh, add=True)

                        jp = j - 1

                        @pl.when(jnp.logical_and(jp >= 0, jp <= 31))
                        def _():
                            pltpu.make_async_copy(
                                gbufs[hp], ysh.at[dstb.at[jp]], ssems[hp]).wait()
                            pltpu.async_copy(table.at[srcb.at[jp + 8]],
                                             gbufs[hp], gsems[hp])

                for h in range(8):
                    pltpu.make_async_copy(gbufs[h], ysh.at[dstb.at[32 + h]],
                                          ssems[h]).wait()

            plsc.subcore_barrier()

            @pl.loop(0, 20)
            def _(i):
                pltpu.sync_copy(ysh.at[pl.ds(t * 640 + i * 32, 32)],
                                out_hbm.at[s, pl.ds(t * 640 + i * 32, 32)])

            plsc.subcore_barrier()

    return k(proj6, src4, dst4, normp)


def _tc_proj(xt, m):
    def body(xa_ref, xb_ref, m_ref, o_ref):
        a = jnp.dot(xa_ref[0], m_ref[...], preferred_element_type=jnp.float32)
        b = jnp.dot(xb_ref[0], m_ref[...], preferred_element_type=jnp.float32)
        o_ref[0] = jnp.concatenate([a, b], axis=1)

    return pl.pallas_call(
        body,
        grid=(NSLICE, NBLK),
        in_specs=[
            pl.BlockSpec((1, BN, DIN), lambda s, i: (2 * s, i, 0)),
            pl.BlockSpec((1, BN, DIN), lambda s, i: (2 * s + 1, i, 0)),
            pl.BlockSpec((DIN, K2), lambda s, i: (0, 0)),
        ],
        out_specs=pl.BlockSpec((1, BN, 128), lambda s, i: (s, i, 0)),
        out_shape=jax.ShapeDtypeStruct((NSLICE, NPAD, 128), jnp.float32),
    )(xt, xt, m)


def _tc_final(yagg, proj6, invdeg, bias128, probs2, W_out, b_out):
    def body(y_ref, p_ref, d_ref, b_ref, pr_ref, wo_ref, bo_ref, o_ref):
        dq = d_ref[...]
        acc = jnp.zeros((BN, DH), jnp.float32)
        for s in range(NSLICE):
            tfull = y_ref[s] + dq * p_ref[s] + b_ref[...]
            for half in range(2):
                p = 2 * s + half
                z = jax.nn.sigmoid(tfull[:, half * K2:half * K2 + DH])
                ht = jnp.tanh(tfull[:, half * K2 + DH:half * K2 + 2 * DH])
                acc = acc + pr_ref[0, p] * ((1.0 - z) * ht)
        o_ref[...] = jnp.dot(jnp.maximum(acc, 0.0), wo_ref[...],
                             preferred_element_type=jnp.float32) + bo_ref[...]

    return pl.pallas_call(
        body,
        grid=(NBLK,),
        in_specs=[
            pl.BlockSpec((NSLICE, BN, 128), lambda i: (0, i, 0)),
            pl.BlockSpec((NSLICE, BN, 128), lambda i: (0, i, 0)),
            pl.BlockSpec((BN, 1), lambda i: (i, 0)),
            pl.BlockSpec((1, 128), lambda i: (0, 0)),
            pl.BlockSpec((1, P), lambda i: (0, 0), memory_space=pltpu.SMEM),
            pl.BlockSpec((DH, P), lambda i: (0, 0)),
            pl.BlockSpec((1, P), lambda i: (0, 0)),
        ],
        out_specs=pl.BlockSpec((BN, P), lambda i: (i, 0)),
        out_shape=jax.ShapeDtypeStruct((NPAD, P), jnp.float32),
    )(yagg, proj6, invdeg, bias128, probs2, W_out, b_out)


def kernel(x, edge_index, edge_attr, Wz, bz, lz_W, lz_b, Wr, br, lr_W, lr_b,
           Wh, bh, lh_W, lh_b, att, W_out, b_out):
    Lz = lz_W[:DH]
    Lh = lh_W[:DH]
    M = jnp.concatenate([Wz @ Lz, Wh @ Lh], axis=1)
    cz = bz @ Lz + lz_b
    ch = bh @ Lh + lh_b
    bias128 = jnp.concatenate([cz, ch, cz, ch]).reshape(1, 128)
    probs2 = jax.nn.softmax(att).reshape(1, P)

    src = edge_index[0]
    dst = edge_index[1]
    pad = EPAD - E
    srcp = jnp.concatenate([src, jnp.zeros((pad,), jnp.int32)])
    dstp = jnp.concatenate([dst, jnp.zeros((pad,), jnp.int32)])
    wp = jnp.concatenate([edge_attr, jnp.zeros((pad,), jnp.float32)])
    src3 = srcp.reshape(32, 40, 128)
    dst3 = dstp.reshape(32, 40, 128)
    w3 = wp.reshape(32, 40, 128)

    degp = _sc_deg(dst3, w3)
    deg = degp[0] + degp[1] + 1.0
    dis = lax.rsqrt(deg)
    invdeg = (1.0 / deg).reshape(NPAD, 1)

    normp = _sc_norm(src3, dst3, w3, dis).reshape(16, 320, 32)

    xt = jnp.transpose(x, (2, 0, 1))
    proj6 = _tc_proj(xt, M)

    yagg = _sc_agg(proj6, srcp.reshape(16, 320, 32),
                   dstp.reshape(16, 320, 32), normp)

    out = _tc_final(yagg, proj6, invdeg, bias128, probs2, W_out,
                    b_out.reshape(1, P))
    return out[:N]


# speedup vs baseline: 1.1021x; 1.0107x over previous
import dataclasses
import functools

import jax
import jax.numpy as jnp
from jax import lax
from jax.experimental import pallas as pl
from jax.experimental.pallas import tpu as pltpu
from jax.experimental.pallas import tpu_sc as plsc

N = 10000
NPAD = 10240
E = 160000
EPAD = 163840
DIN = 256
DH = 32
P = 12
K2 = 2 * DH
NSLICE = 6
BN = 512
NBLK = NPAD // BN


def _mesh_():
    return plsc.VectorSubcoreMesh(core_axis_name="core", subcore_axis_name="subcore",
                                  num_cores=2, num_subcores=16)


def _sc_params():
    cp = pltpu.CompilerParams(use_tc_tiling_on_sc=True)
    if "needs_layout_passes" in pltpu.CompilerParams.__dataclass_fields__:
        cp = dataclasses.replace(cp, needs_layout_passes=False)
    return cp


def _sc_deg(dst3, w3):
    @functools.partial(
        pl.kernel,
        out_type=jax.ShapeDtypeStruct((2, NPAD), jnp.float32),
        mesh=_mesh_(),
        scratch_types=[
            pltpu.VMEM((40, 128), jnp.int32),
            pltpu.VMEM((40, 128), jnp.float32),
            pltpu.VMEM((640,), jnp.float32),
            pltpu.VMEM_SHARED((NPAD,), jnp.float32),
        ],
        compiler_params=_sc_params(),
    )
    def k(dst_hbm, w_hbm, out_hbm, idx_v, w_v, zbuf, deg_sh):
        c = lax.axis_index("core")
        t = lax.axis_index("subcore")
        wid = c * 16 + t

        @pl.loop(0, 40)
        def _(i):
            zbuf[pl.ds(i * 16, 16)] = jnp.zeros((16,), jnp.float32)

        pltpu.sync_copy(zbuf, deg_sh.at[pl.ds(t * 640, 640)])
        plsc.subcore_barrier()

        pltpu.sync_copy(dst_hbm.at[wid], idx_v)
        pltpu.sync_copy(w_hbm.at[wid], w_v)

        @pl.loop(0, 40)
        def _(j):
            pltpu.sync_copy(w_v.at[j], deg_sh.at[idx_v.at[j]], add=True)

        plsc.subcore_barrier()
        pltpu.sync_copy(deg_sh.at[pl.ds(t * 640, 640)],
                        out_hbm.at[c, pl.ds(t * 640, 640)])

    return k(dst3, w3)


def _sc_norm(src3, dst3, w3, dis):
    @functools.partial(
        pl.kernel,
        out_type=jax.ShapeDtypeStruct((32, 40, 128), jnp.float32),
        mesh=_mesh_(),
        scratch_types=[
            pltpu.VMEM((40, 128), jnp.int32),
            pltpu.VMEM((40, 128), jnp.int32),
            pltpu.VMEM((40, 128), jnp.float32),
            pltpu.VMEM((NPAD,), jnp.float32),
        ],
        compiler_params=_sc_params(),
    )
    def k(src_hbm, dst_hbm, w_hbm, dis_hbm, out_hbm, sv, dv, wv, disv):
        c = lax.axis_index("core")
        t = lax.axis_index("subcore")
        wid = c * 16 + t

        pltpu.sync_copy(dis_hbm, disv)
        pltpu.sync_copy(src_hbm.at[wid], sv)
        pltpu.sync_copy(dst_hbm.at[wid], dv)
        pltpu.sync_copy(w_hbm.at[wid], wv)

        @pl.loop(0, 40)
        def _(j):
            for v in range(8):
                sl = pl.ds(v * 16, 16)
                s16 = sv[j, sl]
                d16 = dv[j, sl]
                wv[j, sl] = (plsc.load_gather(disv, [s16]) * wv[j, sl]
                             * plsc.load_gather(disv, [d16]))

        pltpu.sync_copy(wv, out_hbm.at[wid])

    return k(src3, dst3, w3, dis)


def _sc_agg(proj6, src4, dst4, normp):
    @functools.partial(
        pl.kernel,
        out_type=jax.ShapeDtypeStruct((NSLICE, NPAD, 128), jnp.float32),
        mesh=_mesh_(),
        scratch_types=[
            pltpu.VMEM((40, 32), jnp.int32),
            pltpu.VMEM((40, 32), jnp.int32),
            pltpu.VMEM((40, 32), jnp.float32),
            pltpu.VMEM((32, 128), jnp.float32),
            pltpu.VMEM((32, 128), jnp.float32),
            pltpu.VMEM((32, 128), jnp.float32),
            pltpu.VMEM((32, 128), jnp.float32),
            pltpu.VMEM((32, 128), jnp.float32),
            pltpu.SemaphoreType.DMA,
            pltpu.SemaphoreType.DMA,
            pltpu.SemaphoreType.DMA,
            pltpu.SemaphoreType.DMA,
            pltpu.SemaphoreType.DMA,
            pltpu.SemaphoreType.DMA,
            pltpu.SemaphoreType.DMA,
            pltpu.SemaphoreType.DMA,
            pltpu.SemaphoreType.DMA,
            pltpu.SemaphoreType.DMA,
            pltpu.VMEM_SHARED((NPAD, 128), jnp.float32),
        ],
        compiler_params=_sc_params(),
    )
    def k(proj_hbm, src_hbm, dst_hbm, norm_hbm, out_hbm,
          srcb, dstb, normb, g0, g1, g2, g3, g4,
          sg0, sg1, sg2, sg3, sg4, ss0, ss1, ss2, ss3, ss4, ysh):
        c = lax.axis_index("core")
        t = lax.axis_index("subcore")

        gbufs = (g0, g1, g2, g3, g4)
        gsems = (sg0, sg1, sg2, sg3, sg4)
        ssems = (ss0, ss1, ss2, ss3, ss4)

        @pl.loop(0, 3)
        def _(sp):
            s = c * 3 + sp

            @pl.loop(0, 32)
            def _(i):
                for v in range(8):
                    g0[i, pl.ds(v * 16, 16)] = jnp.zeros((16,), jnp.float32)

            @pl.loop(0, 20)
            def _(i):
                pltpu.sync_copy(g0, ysh.at[pl.ds(t * 640 + i * 32, 32)])

            plsc.subcore_barrier()

            table = proj_hbm.at[s]

            @pl.loop(0, 8)
            def _(b):
                pltpu.sync_copy(src_hbm.at[t, pl.ds(b * 40, 40)], srcb)
                for h in range(5):
                    pltpu.async_copy(table.at[srcb.at[h]], gbufs[h], gsems[h])
                pltpu.sync_copy(dst_hbm.at[t, pl.ds(b * 40, 40)], dstb)
                pltpu.sync_copy(norm_hbm.at[t, pl.ds(b * 40, 40)], normb)

                @pl.loop(0, 8)
                def _(q):
                    for h in range(5):
                        gb, sgh, ssh = gbufs[h], gsems[h], ssems[h]
                        hp = (h + 4) % 5
                        j = q * 5 + h
                        pltpu.make_async_copy(table.at[srcb.at[j]], gb, sgh).wait()

                        @pl.loop(0, 2)
                        def _(i16):
                            nv16 = normb[j, pl.ds(i16 * 16, 16)]
                            for l in range(16):
                                i = i16 * 16 + l
                                nv = nv16[l]
                                for v in range(8):
                                    gb[i, pl.ds(v * 16, 16)] = (
                                        gb[i, pl.ds(v * 16, 16)] * nv)

                        pltpu.async_copy(gb, ysh.at[dstb.at[j]], ssh, add=True)

                        jp = j - 1

                        @pl.when(jnp.logical_and(jp >= 0, jp <= 34))
                        def _():
                            pltpu.make_async_copy(
                                gbufs[hp], ysh.at[dstb.at[jp]], ssems[hp]).wait()
                            pltpu.async_copy(table.at[srcb.at[jp + 5]],
                                             gbufs[hp], gsems[hp])

                for h in range(5):
                    pltpu.make_async_copy(gbufs[h], ysh.at[dstb.at[35 + h]],
                                          ssems[h]).wait()

            plsc.subcore_barrier()

            @pl.loop(0, 20)
            def _(i):
                pltpu.sync_copy(ysh.at[pl.ds(t * 640 + i * 32, 32)],
                                out_hbm.at[s, pl.ds(t * 640 + i * 32, 32)])

            plsc.subcore_barrier()

    return k(proj6, src4, dst4, normp)


def _tc_proj(xt, m):
    def body(xa_ref, xb_ref, m_ref, o_ref):
        a = jnp.dot(xa_ref[0], m_ref[...], preferred_element_type=jnp.float32)
        b = jnp.dot(xb_ref[0], m_ref[...], preferred_element_type=jnp.float32)
        o_ref[0] = jnp.concatenate([a, b], axis=1)

    return pl.pallas_call(
        body,
        grid=(NSLICE, NBLK),
        in_specs=[
            pl.BlockSpec((1, BN, DIN), lambda s, i: (2 * s, i, 0)),
            pl.BlockSpec((1, BN, DIN), lambda s, i: (2 * s + 1, i, 0)),
            pl.BlockSpec((DIN, K2), lambda s, i: (0, 0)),
        ],
        out_specs=pl.BlockSpec((1, BN, 128), lambda s, i: (s, i, 0)),
        out_shape=jax.ShapeDtypeStruct((NSLICE, NPAD, 128), jnp.float32),
    )(xt, xt, m)


def _tc_final(yagg, proj6, invdeg, bias128, probs2, W_out, b_out):
    def body(y_ref, p_ref, d_ref, b_ref, pr_ref, wo_ref, bo_ref, o_ref):
        dq = d_ref[...]
        acc = jnp.zeros((BN, DH), jnp.float32)
        for s in range(NSLICE):
            tfull = y_ref[s] + dq * p_ref[s] + b_ref[...]
            for half in range(2):
                p = 2 * s + half
                z = jax.nn.sigmoid(tfull[:, half * K2:half * K2 + DH])
                ht = jnp.tanh(tfull[:, half * K2 + DH:half * K2 + 2 * DH])
                acc = acc + pr_ref[0, p] * ((1.0 - z) * ht)
        o_ref[...] = jnp.dot(jnp.maximum(acc, 0.0), wo_ref[...],
                             preferred_element_type=jnp.float32) + bo_ref[...]

    return pl.pallas_call(
        body,
        grid=(NBLK,),
        in_specs=[
            pl.BlockSpec((NSLICE, BN, 128), lambda i: (0, i, 0)),
            pl.BlockSpec((NSLICE, BN, 128), lambda i: (0, i, 0)),
            pl.BlockSpec((BN, 1), lambda i: (i, 0)),
            pl.BlockSpec((1, 128), lambda i: (0, 0)),
            pl.BlockSpec((1, P), lambda i: (0, 0), memory_space=pltpu.SMEM),
            pl.BlockSpec((DH, P), lambda i: (0, 0)),
            pl.BlockSpec((1, P), lambda i: (0, 0)),
        ],
        out_specs=pl.BlockSpec((BN, P), lambda i: (i, 0)),
        out_shape=jax.ShapeDtypeStruct((NPAD, P), jnp.float32),
    )(yagg, proj6, invdeg, bias128, probs2, W_out, b_out)


def kernel(x, edge_index, edge_attr, Wz, bz, lz_W, lz_b, Wr, br, lr_W, lr_b,
           Wh, bh, lh_W, lh_b, att, W_out, b_out):
    Lz = lz_W[:DH]
    Lh = lh_W[:DH]
    M = jnp.concatenate([Wz @ Lz, Wh @ Lh], axis=1)
    cz = bz @ Lz + lz_b
    ch = bh @ Lh + lh_b
    bias128 = jnp.concatenate([cz, ch, cz, ch]).reshape(1, 128)
    probs2 = jax.nn.softmax(att).reshape(1, P)

    src = edge_index[0]
    dst = edge_index[1]
    pad = EPAD - E
    srcp = jnp.concatenate([src, jnp.zeros((pad,), jnp.int32)])
    dstp = jnp.concatenate([dst, jnp.zeros((pad,), jnp.int32)])
    wp = jnp.concatenate([edge_attr, jnp.zeros((pad,), jnp.float32)])
    src3 = srcp.reshape(32, 40, 128)
    dst3 = dstp.reshape(32, 40, 128)
    w3 = wp.reshape(32, 40, 128)

    degp = _sc_deg(dst3, w3)
    deg = degp[0] + degp[1] + 1.0
    dis = lax.rsqrt(deg)
    invdeg = (1.0 / deg).reshape(NPAD, 1)

    normp = _sc_norm(src3, dst3, w3, dis).reshape(16, 320, 32)

    xt = jnp.transpose(x, (2, 0, 1))
    proj6 = _tc_proj(xt, M)

    yagg = _sc_agg(proj6, srcp.reshape(16, 320, 32),
                   dstp.reshape(16, 320, 32), normp)

    out = _tc_final(yagg, proj6, invdeg, bias128, probs2, W_out,
                    b_out.reshape(1, P))
    return out[:N]
